# baseline (speedup 1.0000x reference)
import jax
import jax.numpy as jnp
from jax.experimental import pallas as pl
from jax.experimental.pallas import tpu as pltpu

BM = 400


def _leaky(x):
    return jnp.where(x >= 0, x, 0.01 * x)


def _xwt(x, w):
    return jax.lax.dot_general(
        x, w, (((1,), (1,)), ((), ())), preferred_element_type=jnp.float32
    )


def _agg_kernel(a_ref, ego_ref, w1_ref, b1_ref, w2_ref, b2_ref, out_ref,
                ego_bf_ref):
    i = pl.program_id(0)

    @pl.when(i == 0)
    def _cast_ego():
        ego_bf_ref[...] = ego_ref[...].astype(jnp.bfloat16)

    a_bf = a_ref[...].astype(jnp.bfloat16)
    side = jnp.dot(a_bf, ego_bf_ref[...], preferred_element_type=jnp.float32)
    ego_row = ego_ref[pl.ds(i * BM, BM), :]
    sum_e = _leaky(_xwt(ego_row + side, w1_ref[...]) + b1_ref[...])
    bi_e = _leaky(_xwt(ego_row * side, w2_ref[...]) + b2_ref[...])
    out_ref[...] = sum_e + bi_e


@jax.jit
def kernel(ego_embeddings, A_in, W1, b1, W2, b2):
    n, d = ego_embeddings.shape
    nm = n // BM
    b1r = b1.reshape(1, d)
    b2r = b2.reshape(1, d)

    out = pl.pallas_call(
        _agg_kernel,
        grid=(nm,),
        in_specs=[
            pl.BlockSpec((BM, n), lambda i: (i, 0)),
            pl.BlockSpec((n, d), lambda i: (0, 0)),
            pl.BlockSpec((d, d), lambda i: (0, 0)),
            pl.BlockSpec((1, d), lambda i: (0, 0)),
            pl.BlockSpec((d, d), lambda i: (0, 0)),
            pl.BlockSpec((1, d), lambda i: (0, 0)),
        ],
        out_specs=pl.BlockSpec((BM, d), lambda i: (i, 0)),
        out_shape=jax.ShapeDtypeStruct((n, d), jnp.float32),
        scratch_shapes=[pltpu.VMEM((n, d), jnp.bfloat16)],
        compiler_params=pltpu.CompilerParams(
            dimension_semantics=("parallel",),
        ),
    )(A_in, ego_embeddings, W1, b1r, W2, b2r)
    return out


# speedup vs baseline: 1.0010x; 1.0010x over previous
import jax
import jax.numpy as jnp
from jax.experimental import pallas as pl
from jax.experimental.pallas import tpu as pltpu

BM = 400


def _leaky(x):
    return jnp.where(x >= 0, x, 0.01 * x)


def _xwt(x, w):
    return jax.lax.dot_general(
        x, w, (((1,), (1,)), ((), ())), preferred_element_type=jnp.float32
    )


def _agg_kernel(a_ref, ego_ref, w1_ref, b1_ref, w2_ref, b2_ref, out_ref):
    i = pl.program_id(0)
    side = jnp.dot(a_ref[...], ego_ref[...],
                   preferred_element_type=jnp.float32)
    ego_row = ego_ref[pl.ds(i * BM, BM), :]
    sum_e = _leaky(_xwt(ego_row + side, w1_ref[...]) + b1_ref[...])
    bi_e = _leaky(_xwt(ego_row * side, w2_ref[...]) + b2_ref[...])
    out_ref[...] = sum_e + bi_e


@jax.jit
def kernel(ego_embeddings, A_in, W1, b1, W2, b2):
    n, d = ego_embeddings.shape
    nm = n // BM
    b1r = b1.reshape(1, d)
    b2r = b2.reshape(1, d)

    out = pl.pallas_call(
        _agg_kernel,
        grid=(nm,),
        in_specs=[
            pl.BlockSpec((BM, n), lambda i: (i, 0)),
            pl.BlockSpec((n, d), lambda i: (0, 0)),
            pl.BlockSpec((d, d), lambda i: (0, 0)),
            pl.BlockSpec((1, d), lambda i: (0, 0)),
            pl.BlockSpec((d, d), lambda i: (0, 0)),
            pl.BlockSpec((1, d), lambda i: (0, 0)),
        ],
        out_specs=pl.BlockSpec((BM, d), lambda i: (i, 0)),
        out_shape=jax.ShapeDtypeStruct((n, d), jnp.float32),
        compiler_params=pltpu.CompilerParams(
            dimension_semantics=("parallel",),
        ),
    )(A_in, ego_embeddings, W1, b1r, W2, b2r)
    return out
